# baseline (speedup 1.0000x reference)
import functools

import jax
import jax.numpy as jnp
from jax import lax
from jax.experimental import pallas as pl
from jax.experimental.pallas import tpu as pltpu
from jax.experimental.pallas import tpu_sc as plsc


@functools.cache
def _make_gather(B, V, D):
    info = plsc.get_sparse_core_info()
    NW = info.num_cores * info.num_subcores
    b_per_w = B // NW
    mesh = plsc.VectorSubcoreMesh(core_axis_name="c", subcore_axis_name="s")

    @functools.partial(
        pl.kernel,
        mesh=mesh,
        out_type=jax.ShapeDtypeStruct((B, D), jnp.float32),
        scratch_types=[
            pltpu.VMEM((b_per_w,), jnp.int32),
            pltpu.SemaphoreType.DMA,
        ],
    )
    def k(idx_hbm, table_hbm, out_hbm, idx_v, sem):
        wid = lax.axis_index("s") * info.num_cores + lax.axis_index("c")
        base = wid * b_per_w
        pltpu.sync_copy(idx_hbm.at[pl.ds(base, b_per_w)], idx_v)

        def body(g, carry):
            v = idx_v[pl.ds(g * info.num_lanes, info.num_lanes)]
            for j in range(info.num_lanes):
                r = v[j]
                pltpu.async_copy(
                    table_hbm.at[pl.ds(r, 1)],
                    out_hbm.at[pl.ds(base + g * info.num_lanes + j, 1)],
                    sem,
                )
            return carry

        lax.fori_loop(0, b_per_w // info.num_lanes, body, 0)
        pltpu.make_async_copy(
            table_hbm.at[pl.ds(0, b_per_w)],
            out_hbm.at[pl.ds(base, b_per_w)],
            sem,
        ).wait()

    return k


def kernel(x, table):
    B = x.shape[0]
    V, D = table.shape
    return _make_gather(B, V, D)(x.astype(jnp.int32), table)
